# baseline (speedup 1.0000x reference)
import functools

import jax
import jax.numpy as jnp
from jax import lax
from jax.experimental import pallas as pl
from jax.experimental.pallas import tpu as pltpu
from jax.experimental.pallas import tpu_sc as plsc

_BATCH = 4096
_SEQ = 200
_EMB = 64
_D1 = 16
_NC = 2
_NS = 16
_NW = _NC * _NS
_RPW = _BATCH // _NW
_C0 = 104
_C1 = _SEQ - _C0

_LANES = 16
_NCH = _EMB // _LANES


def _pool_body(idx_hbm, table_hbm, out_hbm, idx_v, rows_v, out_v, sem):
    wid = lax.axis_index("s") * _NC + lax.axis_index("c")
    base = wid * _RPW
    pltpu.sync_copy(idx_hbm.at[pl.ds(base, _RPW), :], idx_v)

    def row(r, carry):
        cp0 = pltpu.async_copy(
            table_hbm.at[idx_v.at[r, pl.ds(0, _C0)]],
            rows_v.at[pl.ds(0, _C0), :], sem)
        cp1 = pltpu.async_copy(
            table_hbm.at[idx_v.at[r, pl.ds(_C0, _C1)]],
            rows_v.at[pl.ds(_C0, _C1), :], sem)
        cp0.wait()
        cp1.wait()

        def acc_body(j, acc):
            return tuple(
                acc[c] + rows_v[j, pl.ds(c * _LANES, _LANES)]
                for c in range(_NCH))

        zeros = tuple(jnp.zeros((_LANES,), jnp.float32) for _ in range(_NCH))
        acc = lax.fori_loop(0, _SEQ, acc_body, zeros)
        for c in range(_NCH):
            out_v[r, pl.ds(c * _LANES, _LANES)] = acc[c] * (1.0 / _SEQ)
        return carry

    lax.fori_loop(0, _RPW, row, 0)
    pltpu.sync_copy(out_v, out_hbm.at[pl.ds(base, _RPW), :])


def _pool(idx, table):
    mesh = plsc.VectorSubcoreMesh(core_axis_name="c", subcore_axis_name="s")
    f = pl.kernel(
        _pool_body,
        out_type=jax.ShapeDtypeStruct((_BATCH, _EMB), jnp.float32),
        mesh=mesh,
        scratch_types=[
            pltpu.VMEM((_RPW, _SEQ), jnp.int32),
            pltpu.VMEM((_SEQ, _EMB), jnp.float32),
            pltpu.VMEM((_RPW, _EMB), jnp.float32),
            pltpu.SemaphoreType.DMA,
        ],
        compiler_params=pltpu.CompilerParams(use_tc_tiling_on_sc=False),
    )
    return f(idx, table)


def _mlp_body(pooled_ref, w1_ref, b1_ref, w2_ref, b2_ref, out_ref):
    h = jnp.dot(pooled_ref[...], w1_ref[...],
                preferred_element_type=jnp.float32) + b1_ref[...]
    h = jnp.maximum(h, 0.0)
    z = jnp.dot(h, w2_ref[...], preferred_element_type=jnp.float32)
    z = z + b2_ref[...]
    out_ref[...] = 1.0 / (1.0 + jnp.exp(-z))


def kernel(inputs, emb_table, W1, b1, W2, b2):
    idx = inputs.astype(jnp.int32)
    pooled = _pool(idx, emb_table)
    out = pl.pallas_call(
        _mlp_body,
        out_shape=jax.ShapeDtypeStruct((_BATCH, 1), jnp.float32),
    )(pooled, W1, b1.reshape(1, _D1), W2, b2.reshape(1, 1))
    return out


# speedup vs baseline: 1.1711x; 1.1711x over previous
import functools

import jax
import jax.numpy as jnp
from jax import lax
from jax.experimental import pallas as pl
from jax.experimental.pallas import tpu as pltpu
from jax.experimental.pallas import tpu_sc as plsc

_BATCH = 4096
_SEQ = 200
_EMB = 64
_D1 = 16
_NC = 2
_NS = 16
_NW = _NC * _NS
_RPW = _BATCH // _NW
_C0 = 104
_C1 = _SEQ - _C0

_LANES = 16
_NCH = _EMB // _LANES


def _pool_body(idx_hbm, table_hbm, out_hbm, idx_v, rows_a, rows_b,
               out_v, sem_a, sem_b):
    wid = lax.axis_index("s") * _NC + lax.axis_index("c")
    base = wid * _RPW
    pltpu.sync_copy(idx_hbm.at[pl.ds(base, _RPW), :], idx_v)

    bufs = (rows_a, rows_b)
    sems = (sem_a, sem_b)

    def issue(r, buf, sem):
        pltpu.async_copy(
            table_hbm.at[idx_v.at[r, pl.ds(0, _C0)]],
            buf.at[pl.ds(0, _C0), :], sem)
        pltpu.async_copy(
            table_hbm.at[idx_v.at[r, pl.ds(_C0, _C1)]],
            buf.at[pl.ds(_C0, _C1), :], sem)

    def drain(r, buf, sem):
        pltpu.make_async_copy(
            table_hbm.at[idx_v.at[r, pl.ds(0, _C0)]],
            buf.at[pl.ds(0, _C0), :], sem).wait()
        pltpu.make_async_copy(
            table_hbm.at[idx_v.at[r, pl.ds(_C0, _C1)]],
            buf.at[pl.ds(_C0, _C1), :], sem).wait()

    def consume(r, buf):
        def acc_body(t, carry):
            a, b = carry
            j = t * 4
            for q in range(4):
                src = tuple(
                    buf[j + q, pl.ds(c * _LANES, _LANES)]
                    for c in range(_NCH))
                if q % 2 == 0:
                    a = tuple(a[c] + src[c] for c in range(_NCH))
                else:
                    b = tuple(b[c] + src[c] for c in range(_NCH))
            return a, b

        zeros = tuple(jnp.zeros((_LANES,), jnp.float32)
                      for _ in range(_NCH))
        a, b = lax.fori_loop(0, _SEQ // 4, acc_body, (zeros, zeros))
        for c in range(_NCH):
            out_v[r, pl.ds(c * _LANES, _LANES)] = \
                (a[c] + b[c]) * (1.0 / _SEQ)

    issue(0, bufs[0], sems[0])

    def pair(p, carry):
        for par in (0, 1):
            r = p * 2 + par
            nxt = r + 1

            @pl.when(nxt < _RPW)
            def _():
                issue(nxt, bufs[1 - par], sems[1 - par])

            drain(r, bufs[par], sems[par])
            consume(r, bufs[par])
        return carry

    lax.fori_loop(0, _RPW // 2, pair, 0)
    pltpu.sync_copy(out_v, out_hbm.at[pl.ds(base, _RPW), :])


def _pool(idx, table):
    mesh = plsc.VectorSubcoreMesh(core_axis_name="c", subcore_axis_name="s")
    f = pl.kernel(
        _pool_body,
        out_type=jax.ShapeDtypeStruct((_BATCH, _EMB), jnp.float32),
        mesh=mesh,
        scratch_types=[
            pltpu.VMEM((_RPW, _SEQ), jnp.int32),
            pltpu.VMEM((_SEQ, _EMB), jnp.float32),
            pltpu.VMEM((_SEQ, _EMB), jnp.float32),
            pltpu.VMEM((_RPW, _EMB), jnp.float32),
            pltpu.SemaphoreType.DMA,
            pltpu.SemaphoreType.DMA,
        ],
        compiler_params=pltpu.CompilerParams(use_tc_tiling_on_sc=False),
    )
    return f(idx, table)


def _mlp_body(pooled_ref, w1_ref, b1_ref, w2_ref, b2_ref, out_ref):
    h = jnp.dot(pooled_ref[...], w1_ref[...],
                preferred_element_type=jnp.float32) + b1_ref[...]
    h = jnp.maximum(h, 0.0)
    z = jnp.dot(h, w2_ref[...], preferred_element_type=jnp.float32)
    z = z + b2_ref[...]
    out_ref[...] = 1.0 / (1.0 + jnp.exp(-z))


def kernel(inputs, emb_table, W1, b1, W2, b2):
    idx = inputs.astype(jnp.int32)
    pooled = _pool(idx, emb_table)
    out = pl.pallas_call(
        _mlp_body,
        out_shape=jax.ShapeDtypeStruct((_BATCH, 1), jnp.float32),
    )(pooled, W1, b1.reshape(1, _D1), W2, b2.reshape(1, 1))
    return out
